# baseline (speedup 1.0000x reference)
import functools

import jax
import jax.numpy as jnp
from jax.experimental import pallas as pl
from jax.experimental.pallas import tpu as pltpu

N = 32 * 32 * 32
D = 64
H = 256
E = 8
K = 2

BLK = 2048


def _moe_block(x_ref, wr_ref, w1_ref, b1_ref, w2_ref, b2_ref, o_ref):
    x = x_ref[...]
    logits = jax.lax.dot_general(
        x, wr_ref[...], (((1,), (0,)), ((), ())),
        preferred_element_type=jnp.float32)

    idx = jax.lax.broadcasted_iota(jnp.int32, logits.shape, 1)
    m1 = jnp.max(logits, axis=1, keepdims=True)
    i1 = jnp.min(jnp.where(logits == m1, idx, E), axis=1, keepdims=True)
    l2 = jnp.where(idx == i1, -jnp.inf, logits)
    m2 = jnp.max(l2, axis=1, keepdims=True)
    i2 = jnp.min(jnp.where(l2 == m2, idx, E), axis=1, keepdims=True)
    g2 = 1.0 / (1.0 + jnp.exp(m1 - m2))
    g1 = 1.0 - g2

    acc = x
    for e in range(E):
        h = jax.lax.dot_general(
            x, w1_ref[e], (((1,), (0,)), ((), ())),
            preferred_element_type=jnp.float32) + b1_ref[e][None, :]
        h = jax.nn.gelu(h)
        y = jax.lax.dot_general(
            h, w2_ref[e], (((1,), (0,)), ((), ())),
            preferred_element_type=jnp.float32) + b2_ref[e][None, :]
        ge = g1 * (i1 == e) + g2 * (i2 == e)
        acc = acc + ge * y
    o_ref[...] = acc


@jax.jit
def kernel(states, W_router, W1, b1, W2, b2):
    grid = (N // BLK,)
    return pl.pallas_call(
        _moe_block,
        grid=grid,
        in_specs=[
            pl.BlockSpec((BLK, D), lambda i: (i, 0)),
            pl.BlockSpec((D, E), lambda i: (0, 0)),
            pl.BlockSpec((E, D, H), lambda i: (0, 0, 0)),
            pl.BlockSpec((E, H), lambda i: (0, 0)),
            pl.BlockSpec((E, H, D), lambda i: (0, 0, 0)),
            pl.BlockSpec((E, D), lambda i: (0, 0)),
        ],
        out_specs=pl.BlockSpec((BLK, D), lambda i: (i, 0)),
        out_shape=jax.ShapeDtypeStruct((N, D), jnp.float32),
        compiler_params=pltpu.CompilerParams(
            dimension_semantics=("arbitrary",),
        ),
    )(states, W_router, W1, b1, W2, b2)
